# Initial kernel scaffold; baseline (speedup 1.0000x reference)
import jax
import jax.numpy as jnp
from jax.experimental import pallas as pl


def kernel(semantic_ids, tables):
    raise NotImplementedError("write your pallas kernel here")


# speedup vs baseline: 2.6606x; 2.6606x over previous
import functools

import jax
import jax.numpy as jnp
from jax import lax
from jax.experimental import pallas as pl
from jax.experimental.pallas import tpu as pltpu
from jax.experimental.pallas import tpu_sc as plsc


def _build_sc_kernel(N, L, K, D, n_workers):
    C = 32
    assert N % (n_workers * C) == 0
    per_worker = N // n_workers
    n_chunks = per_worker // C
    LC = L * C

    mesh = plsc.VectorSubcoreMesh(core_axis_name="c", subcore_axis_name="s")

    @functools.partial(
        pl.kernel,
        mesh=mesh,
        out_type=jax.ShapeDtypeStruct((N, D), jnp.float32),
        scratch_types=[
            pltpu.VMEM((LC,), jnp.int32),
            pltpu.VMEM((LC,), jnp.int32),
            pltpu.VMEM((LC, D), jnp.float32),
            pltpu.VMEM((C, D), jnp.float32),
            pltpu.SemaphoreType.DMA,
        ],
    )
    def sc_kernel(ids_hbm, tab_hbm, out_hbm, ids_v, idx_v, gbuf, obuf, sem):
        wid = lax.axis_index("s") * 2 + lax.axis_index("c")
        wbase = wid * per_worker

        off = (lax.iota(jnp.int32, 16) % L) * K
        inv_l = jnp.float32(1.0 / L)

        def chunk_body(g, carry):
            tok0 = wbase + g * C
            pltpu.sync_copy(ids_hbm.at[pl.ds(tok0 * L, LC)], ids_v)
            for v in range(LC // 16):
                sl = pl.ds(16 * v, 16)
                idx_v[sl] = ids_v[sl] + off
            pltpu.async_copy(tab_hbm.at[idx_v], gbuf, sem).wait()

            def red_body(t, c2):
                r0 = L * t
                for k in range(D // 16):
                    sl = pl.ds(16 * k, 16)
                    a = gbuf[r0, sl] + gbuf[r0 + 1, sl]
                    b = gbuf[r0 + 2, sl] + gbuf[r0 + 3, sl]
                    obuf[t, sl] = (a + b) * inv_l
                return c2

            lax.fori_loop(0, C, red_body, 0, unroll=False)
            pltpu.sync_copy(obuf, out_hbm.at[pl.ds(tok0, C)])
            return carry

        lax.fori_loop(0, n_chunks, chunk_body, 0, unroll=False)

    return sc_kernel


def kernel(semantic_ids, tables):
    B, S, L = semantic_ids.shape
    Lt, K, D = tables.shape
    assert L == Lt == 4
    N = B * S
    ids_flat = semantic_ids.astype(jnp.int32).reshape(N * L)
    tab_flat = tables.reshape(L * K, D)
    out = _build_sc_kernel(N, L, K, D, 32)(ids_flat, tab_flat)
    return out.reshape(B, S, D)


# speedup vs baseline: 4.1759x; 1.5695x over previous
import functools

import jax
import jax.numpy as jnp
from jax import lax
from jax.experimental import pallas as pl
from jax.experimental.pallas import tpu as pltpu
from jax.experimental.pallas import tpu_sc as plsc


def _build_sc_kernel(N, L, K, D, n_workers):
    C = 32
    LC = L * C
    assert N % (n_workers * C) == 0
    per_worker = N // n_workers
    n_chunks = per_worker // C
    n_pairs = n_chunks // 2
    assert n_chunks % 2 == 0 and n_chunks >= 6

    mesh = plsc.VectorSubcoreMesh(core_axis_name="c", subcore_axis_name="s")

    @functools.partial(
        pl.kernel,
        mesh=mesh,
        out_type=jax.ShapeDtypeStruct((N, D), jnp.float32),
        scratch_types=[
            pltpu.VMEM((LC,), jnp.int32),
            pltpu.VMEM((LC,), jnp.int32),
            pltpu.VMEM((LC,), jnp.int32),
            pltpu.VMEM((LC,), jnp.int32),
            pltpu.VMEM((LC, D), jnp.float32),
            pltpu.VMEM((LC, D), jnp.float32),
            pltpu.VMEM((C, D), jnp.float32),
            pltpu.VMEM((C, D), jnp.float32),
            pltpu.SemaphoreType.DMA,
            pltpu.SemaphoreType.DMA,
            pltpu.SemaphoreType.DMA,
            pltpu.SemaphoreType.DMA,
            pltpu.SemaphoreType.DMA,
            pltpu.SemaphoreType.DMA,
        ],
    )
    def sc_kernel(ids_hbm, tab_hbm, out_hbm, idsb0, idsb1, idxb0, idxb1,
                  gbuf0, gbuf1, obuf0, obuf1, sid0, sid1, sg0, sg1, so0, so1):
        wid = lax.axis_index("s") * 2 + lax.axis_index("c")
        wbase = wid * per_worker

        off = (lax.iota(jnp.int32, 16) % L) * K
        inv_l = jnp.float32(1.0 / L)

        idsb = (idsb0, idsb1)
        idxb = (idxb0, idxb1)
        gbuf = (gbuf0, gbuf1)
        obuf = (obuf0, obuf1)
        sid = (sid0, sid1)
        sg = (sg0, sg1)
        so = (so0, so1)

        def ids_slice(g):
            return ids_hbm.at[pl.ds((wbase + g * C) * L, LC)]

        def out_slice(g):
            return out_hbm.at[pl.ds(wbase + g * C, C)]

        def build_idx(e):
            for v in range(LC // 16):
                sl = pl.ds(16 * v, 16)
                idxb[e][sl] = idsb[e][sl] + off

        def gather_copy(e):
            return pltpu.make_async_copy(tab_hbm.at[idxb[e]], gbuf[e], sg[e])

        def reduce_chunk(e):
            def red_body(t, c2):
                r0 = L * t
                for k in range(D // 16):
                    sl = pl.ds(16 * k, 16)
                    a = gbuf[e][r0, sl] + gbuf[e][r0 + 1, sl]
                    b = gbuf[e][r0 + 2, sl] + gbuf[e][r0 + 3, sl]
                    obuf[e][t, sl] = (a + b) * inv_l
                return c2

            lax.fori_loop(0, C, red_body, 0, unroll=2)

        def step(g, e, wait_out):
            gather_copy(e).wait()

            @pl.when(g + 2 < n_chunks)
            def _():
                pltpu.make_async_copy(ids_slice(g + 2), idsb[e], sid[e]).wait()
                build_idx(e)

            @pl.when(g + 4 < n_chunks)
            def _():
                pltpu.make_async_copy(ids_slice(g + 4), idsb[e], sid[e]).start()

            if wait_out:
                pltpu.make_async_copy(obuf[e], out_slice(g), so[e]).wait()
            reduce_chunk(e)

            @pl.when(g + 2 < n_chunks)
            def _():
                gather_copy(e).start()

            pltpu.make_async_copy(obuf[e], out_slice(g), so[e]).start()

        pltpu.sync_copy(ids_slice(0), idsb0)
        pltpu.sync_copy(ids_slice(1), idsb1)
        build_idx(0)
        gather_copy(0).start()
        build_idx(1)
        gather_copy(1).start()
        pltpu.make_async_copy(ids_slice(2), idsb0, sid0).start()
        pltpu.make_async_copy(ids_slice(3), idsb1, sid1).start()
        step(0, 0, wait_out=False)
        step(1, 1, wait_out=False)

        def body(i, carry):
            step(2 * i, 0, wait_out=True)
            step(2 * i + 1, 1, wait_out=True)
            return carry

        lax.fori_loop(1, n_pairs, body, 0, unroll=False)

        pltpu.make_async_copy(obuf0, out_slice(n_chunks - 2), so0).wait()
        pltpu.make_async_copy(obuf1, out_slice(n_chunks - 1), so1).wait()

    return sc_kernel


def kernel(semantic_ids, tables):
    B, S, L = semantic_ids.shape
    Lt, K, D = tables.shape
    assert L == Lt == 4
    N = B * S
    ids_flat = semantic_ids.astype(jnp.int32).reshape(N * L)
    tab_flat = tables.reshape(L * K, D)
    out = _build_sc_kernel(N, L, K, D, 32)(ids_flat, tab_flat)
    return out.reshape(B, S, D)


# speedup vs baseline: 4.1791x; 1.0008x over previous
import functools

import jax
import jax.numpy as jnp
from jax import lax
from jax.experimental import pallas as pl
from jax.experimental.pallas import tpu as pltpu
from jax.experimental.pallas import tpu_sc as plsc


def _build_sc_kernel(N, L, K, D, n_workers):
    C = 32
    LC = L * C
    assert N % (n_workers * C) == 0
    per_worker = N // n_workers
    n_chunks = per_worker // C
    n_pairs = n_chunks // 2
    assert n_chunks % 2 == 0 and n_chunks >= 6

    mesh = plsc.VectorSubcoreMesh(core_axis_name="c", subcore_axis_name="s")

    @functools.partial(
        pl.kernel,
        mesh=mesh,
        compiler_params=pltpu.CompilerParams(use_tc_tiling_on_sc=True),
        out_type=jax.ShapeDtypeStruct((N, D), jnp.float32),
        scratch_types=[
            pltpu.VMEM((LC,), jnp.int32),
            pltpu.VMEM((LC,), jnp.int32),
            pltpu.VMEM((LC,), jnp.int32),
            pltpu.VMEM((LC,), jnp.int32),
            pltpu.VMEM((LC, D), jnp.float32),
            pltpu.VMEM((LC, D), jnp.float32),
            pltpu.VMEM((C, D), jnp.float32),
            pltpu.VMEM((C, D), jnp.float32),
            pltpu.SemaphoreType.DMA,
            pltpu.SemaphoreType.DMA,
            pltpu.SemaphoreType.DMA,
            pltpu.SemaphoreType.DMA,
            pltpu.SemaphoreType.DMA,
            pltpu.SemaphoreType.DMA,
        ],
    )
    def sc_kernel(ids_hbm, tab_hbm, out_hbm, idsb0, idsb1, idxb0, idxb1,
                  gbuf0, gbuf1, obuf0, obuf1, sid0, sid1, sg0, sg1, so0, so1):
        wid = lax.axis_index("s") * 2 + lax.axis_index("c")
        wbase = wid * per_worker

        off = (lax.iota(jnp.int32, 16) % L) * K
        inv_l = jnp.float32(1.0 / L)

        idsb = (idsb0, idsb1)
        idxb = (idxb0, idxb1)
        gbuf = (gbuf0, gbuf1)
        obuf = (obuf0, obuf1)
        sid = (sid0, sid1)
        sg = (sg0, sg1)
        so = (so0, so1)

        def ids_slice(g):
            return ids_hbm.at[pl.ds((wbase + g * C) * L, LC)]

        def out_slice(g):
            return out_hbm.at[pl.ds(wbase + g * C, C)]

        def build_idx(e):
            for v in range(LC // 16):
                sl = pl.ds(16 * v, 16)
                idxb[e][sl] = idsb[e][sl] + off

        def gather_copy(e):
            return pltpu.make_async_copy(tab_hbm.at[idxb[e]], gbuf[e], sg[e])

        def reduce_chunk(e):
            def red_body(t, c2):
                r0 = L * t
                for k in range(D // 16):
                    sl = pl.ds(16 * k, 16)
                    a = gbuf[e][r0, sl] + gbuf[e][r0 + 1, sl]
                    b = gbuf[e][r0 + 2, sl] + gbuf[e][r0 + 3, sl]
                    obuf[e][t, sl] = (a + b) * inv_l
                return c2

            lax.fori_loop(0, C, red_body, 0, unroll=2)

        def step(g, e, wait_out):
            gather_copy(e).wait()

            @pl.when(g + 2 < n_chunks)
            def _():
                pltpu.make_async_copy(ids_slice(g + 2), idsb[e], sid[e]).wait()
                build_idx(e)

            @pl.when(g + 4 < n_chunks)
            def _():
                pltpu.make_async_copy(ids_slice(g + 4), idsb[e], sid[e]).start()

            if wait_out:
                pltpu.make_async_copy(obuf[e], out_slice(g), so[e]).wait()
            reduce_chunk(e)

            @pl.when(g + 2 < n_chunks)
            def _():
                gather_copy(e).start()

            pltpu.make_async_copy(obuf[e], out_slice(g), so[e]).start()

        pltpu.sync_copy(ids_slice(0), idsb0)
        pltpu.sync_copy(ids_slice(1), idsb1)
        build_idx(0)
        gather_copy(0).start()
        build_idx(1)
        gather_copy(1).start()
        pltpu.make_async_copy(ids_slice(2), idsb0, sid0).start()
        pltpu.make_async_copy(ids_slice(3), idsb1, sid1).start()
        step(0, 0, wait_out=False)
        step(1, 1, wait_out=False)

        def body(i, carry):
            step(2 * i, 0, wait_out=True)
            step(2 * i + 1, 1, wait_out=True)
            return carry

        lax.fori_loop(1, n_pairs, body, 0, unroll=False)

        pltpu.make_async_copy(obuf0, out_slice(n_chunks - 2), so0).wait()
        pltpu.make_async_copy(obuf1, out_slice(n_chunks - 1), so1).wait()

    return sc_kernel


def kernel(semantic_ids, tables):
    B, S, L = semantic_ids.shape
    Lt, K, D = tables.shape
    assert L == Lt == 4
    N = B * S
    ids_flat = semantic_ids.astype(jnp.int32).reshape(N * L)
    tab_flat = tables.reshape(L * K, D)
    out = _build_sc_kernel(N, L, K, D, 32)(ids_flat, tab_flat)
    return out.reshape(B, S, D)
